# baseline (speedup 1.0000x reference)
import functools

import jax
import jax.numpy as jnp
from jax.experimental import pallas as pl

N_NODES = 100000
D = 128
D_GRAPH = 256
NUM_GRAPHS = 1024

BLOCK = 2000
WIN = 64


def _fused_kernel(hv_ref, ids_ref, wg_ref, bg_ref, wp_ref, bp_ref, out_ref):
    step = pl.program_id(0)

    @pl.when(step == 0)
    def _init():
        out_ref[...] = jnp.zeros_like(out_ref)

    hv = hv_ref[...]
    ids = ids_ref[0, 0, :]

    gate_lin = jax.lax.dot_general(
        hv, wg_ref[...], (((1,), (0,)), ((), ())),
        preferred_element_type=jnp.float32)
    gate = jax.nn.sigmoid(gate_lin + bg_ref[0, 0])
    proj = jax.lax.dot_general(
        hv.astype(jnp.bfloat16), wp_ref[...].astype(jnp.bfloat16),
        (((1,), (0,)), ((), ())),
        preferred_element_type=jnp.float32) + bp_ref[...]
    hg = gate * proj

    first = jnp.min(ids)
    last = jnp.max(ids)
    w_lo = first // WIN
    n_win = last // WIN - w_lo + 1

    ids_col = ids[:, None]
    lane = jax.lax.broadcasted_iota(jnp.int32, (BLOCK, WIN), 1)

    hg16 = hg.astype(jnp.bfloat16)

    def body(k, _):
        w0 = (w_lo + k) * WIN
        onehot = (ids_col == (lane + w0)).astype(jnp.bfloat16)
        partial = jax.lax.dot_general(
            onehot, hg16, (((0,), (0,)), ((), ())),
            preferred_element_type=jnp.float32)
        out_ref[pl.ds(w0, WIN), :] += partial
        return 0

    jax.lax.fori_loop(0, n_win, body, 0)


@jax.jit
def kernel(hv, segment_ids, W_gate, b_gate, W_g2g, b_g2g):
    ids = segment_ids.astype(jnp.int32).reshape(N_NODES // BLOCK, 1, BLOCK)
    bg = b_gate.reshape(1, 1)
    bp = b_g2g.reshape(1, D_GRAPH)
    grid = (N_NODES // BLOCK,)
    return pl.pallas_call(
        _fused_kernel,
        grid=grid,
        in_specs=[
            pl.BlockSpec((BLOCK, D), lambda i: (i, 0)),
            pl.BlockSpec((1, 1, BLOCK), lambda i: (i, 0, 0)),
            pl.BlockSpec((D, 1), lambda i: (0, 0)),
            pl.BlockSpec((1, 1), lambda i: (0, 0)),
            pl.BlockSpec((D, D_GRAPH), lambda i: (0, 0)),
            pl.BlockSpec((1, D_GRAPH), lambda i: (0, 0)),
        ],
        out_specs=pl.BlockSpec((NUM_GRAPHS, D_GRAPH), lambda i: (0, 0)),
        out_shape=jax.ShapeDtypeStruct((NUM_GRAPHS, D_GRAPH), jnp.float32),
    )(hv, ids, W_gate, bg, W_g2g, bp)


# speedup vs baseline: 1.1190x; 1.1190x over previous
import functools

import jax
import jax.numpy as jnp
from jax.experimental import pallas as pl

N_NODES = 100000
D = 128
D_GRAPH = 256
NUM_GRAPHS = 1024

BLOCK = 4000
WIN = 64


def _fused_kernel(hv_ref, ids_ref, wg_ref, bg_ref, wp_ref, bp_ref, out_ref):
    step = pl.program_id(0)

    @pl.when(step == 0)
    def _init():
        out_ref[...] = jnp.zeros_like(out_ref)

    hv = hv_ref[...]
    ids = ids_ref[0, 0, :]

    gate_lin = jax.lax.dot_general(
        hv, wg_ref[...], (((1,), (0,)), ((), ())),
        preferred_element_type=jnp.float32)
    gate = jax.nn.sigmoid(gate_lin + bg_ref[0, 0])
    proj = jax.lax.dot_general(
        hv.astype(jnp.bfloat16), wp_ref[...].astype(jnp.bfloat16),
        (((1,), (0,)), ((), ())),
        preferred_element_type=jnp.float32) + bp_ref[...]
    hg = gate * proj

    first = jnp.min(ids)
    last = jnp.max(ids)
    w_lo = first // WIN
    n_win = last // WIN - w_lo + 1

    ids_col = ids[:, None]
    lane = jax.lax.broadcasted_iota(jnp.int32, (BLOCK, WIN), 1)

    hg16 = hg.astype(jnp.bfloat16)

    def body(k, _):
        w0 = (w_lo + k) * WIN
        onehot = (ids_col == (lane + w0)).astype(jnp.bfloat16)
        partial = jax.lax.dot_general(
            onehot, hg16, (((0,), (0,)), ((), ())),
            preferred_element_type=jnp.float32)
        out_ref[pl.ds(w0, WIN), :] += partial
        return 0

    jax.lax.fori_loop(0, n_win, body, 0)


@jax.jit
def kernel(hv, segment_ids, W_gate, b_gate, W_g2g, b_g2g):
    ids = segment_ids.astype(jnp.int32).reshape(N_NODES // BLOCK, 1, BLOCK)
    bg = b_gate.reshape(1, 1)
    bp = b_g2g.reshape(1, D_GRAPH)
    grid = (N_NODES // BLOCK,)
    return pl.pallas_call(
        _fused_kernel,
        grid=grid,
        in_specs=[
            pl.BlockSpec((BLOCK, D), lambda i: (i, 0)),
            pl.BlockSpec((1, 1, BLOCK), lambda i: (i, 0, 0)),
            pl.BlockSpec((D, 1), lambda i: (0, 0)),
            pl.BlockSpec((1, 1), lambda i: (0, 0)),
            pl.BlockSpec((D, D_GRAPH), lambda i: (0, 0)),
            pl.BlockSpec((1, D_GRAPH), lambda i: (0, 0)),
        ],
        out_specs=pl.BlockSpec((NUM_GRAPHS, D_GRAPH), lambda i: (0, 0)),
        out_shape=jax.ShapeDtypeStruct((NUM_GRAPHS, D_GRAPH), jnp.float32),
    )(hv, ids, W_gate, bg, W_g2g, bp)
